# baseline (speedup 1.0000x reference)
import functools

import jax
import jax.numpy as jnp
from jax import lax
from jax.experimental import pallas as pl
from jax.experimental.pallas import tpu as pltpu
from jax.experimental.pallas import tpu_sc as plsc

_D = 16
_LB = 40


def _table_body(wte_ref, w_ref, b_ref, out_ref):
    out_ref[...] = lax.dot_general(
        w_ref[...], wte_ref[...],
        (((1,), (1,)), ((), ())),
        preferred_element_type=jnp.float32,
    ) + b_ref[...]


def _fused_table_t(wte, W, b):
    v, d = wte.shape
    return pl.pallas_call(
        _table_body,
        out_shape=jax.ShapeDtypeStruct((d, v), jnp.float32),
    )(wte, W, b.reshape(d, 1))


@functools.cache
def _make_gather(Bt, L, V, NC, NS):
    NW = NC * NS
    SB = Bt // NW
    NGRP = SB // 16
    assert Bt % NW == 0 and L % _LB == 0 and L % 2 == 0 and SB % 16 == 0
    mesh = plsc.VectorSubcoreMesh(core_axis_name="c", subcore_axis_name="s")

    @functools.partial(
        pl.kernel,
        mesh=mesh,
        compiler_params=pltpu.CompilerParams(needs_layout_passes=False),
        out_type=jax.ShapeDtypeStruct((L * _D, Bt), jnp.float32),
        scratch_types=[
            pltpu.VMEM((_D * V,), jnp.float32),
            pltpu.VMEM((_LB, SB), jnp.int32),
            pltpu.VMEM((2, _D, SB), jnp.float32),
            pltpu.SemaphoreType.DMA,
            pltpu.SemaphoreType.DMA,
        ],
    )
    def gather_kernel(xt_hbm, tab_hbm, out_hbm, tab_v, idx_v, ob_v,
                      sem_o0, sem_o1):
        cid = lax.axis_index("c")
        sid = lax.axis_index("s")
        wid = sid * NC + cid
        b0 = pl.multiple_of(wid * SB, SB)
        out_sems = (sem_o0, sem_o1)

        pltpu.sync_copy(tab_hbm, tab_v)

        def stage_block(l0):
            pltpu.sync_copy(
                xt_hbm.at[pl.ds(l0, _LB), pl.ds(b0, SB)], idx_v)

        def compute_l(l, buf):
            ll = lax.rem(l, _LB)
            obuf = ob_v.at[buf]

            @plsc.parallel_loop(0, SB, step=16, unroll=8)
            def group(o):
                idx = idx_v[ll, pl.ds(o, 16)]
                for f in range(_D):
                    vals = plsc.load_gather(tab_v, [idx + (f * V)])
                    obuf[f, pl.ds(o, 16)] = vals
            pltpu.async_copy(
                ob_v.at[buf],
                out_hbm.at[pl.ds(pl.multiple_of(l * _D, _D), _D),
                           pl.ds(b0, SB)],
                out_sems[buf],
            )

        def drain_out(l, buf):
            pltpu.make_async_copy(
                ob_v.at[buf],
                out_hbm.at[pl.ds(pl.multiple_of(l * _D, _D), _D),
                           pl.ds(b0, SB)],
                out_sems[buf],
            ).wait()

        stage_block(0)
        compute_l(0, 0)
        compute_l(1, 1)

        def body(i, carry):
            l0 = 2 * i

            @pl.when(lax.rem(l0, _LB) == 0)
            def _():
                stage_block(pl.multiple_of(l0, _LB))

            for buf in (0, 1):
                l = l0 + buf
                drain_out(l - 2, buf)
                compute_l(l, buf)
            return carry

        lax.fori_loop(1, L // 2, body, 0)
        drain_out(L - 2, 0)
        drain_out(L - 1, 1)

    return gather_kernel


def kernel(x, wte, W, b):
    tableT = _fused_table_t(wte, W, b)
    Bt, L = x.shape
    info = plsc.get_sparse_core_info()
    xt = x.T.astype(jnp.int32)
    out = _make_gather(Bt, L, wte.shape[0], info.num_cores,
                       info.num_subcores)(xt, tableT.reshape(-1))
    return out.reshape(L, _D, Bt).transpose(2, 0, 1)


# speedup vs baseline: 1.0811x; 1.0811x over previous
import functools

import jax
import jax.numpy as jnp
from jax import lax
from jax.experimental import pallas as pl
from jax.experimental.pallas import tpu as pltpu
from jax.experimental.pallas import tpu_sc as plsc

_D = 16
_LB = 40
_NB = 4


def _table_body(wte_ref, w_ref, b_ref, out_ref):
    out_ref[...] = lax.dot_general(
        w_ref[...], wte_ref[...],
        (((1,), (1,)), ((), ())),
        preferred_element_type=jnp.float32,
    ) + b_ref[...]


def _fused_table_t(wte, W, b):
    v, d = wte.shape
    return pl.pallas_call(
        _table_body,
        out_shape=jax.ShapeDtypeStruct((d, v), jnp.float32),
    )(wte, W, b.reshape(d, 1))


@functools.cache
def _make_gather(Bt, L, V, NC, NS):
    NW = NC * NS
    SB = Bt // NW
    NGRP = SB // 16
    assert Bt % NW == 0 and L % _LB == 0 and SB % 16 == 0
    assert L % _NB == 0 and _LB % _NB == 0
    mesh = plsc.VectorSubcoreMesh(core_axis_name="c", subcore_axis_name="s")

    @functools.partial(
        pl.kernel,
        mesh=mesh,
        compiler_params=pltpu.CompilerParams(needs_layout_passes=False),
        out_type=jax.ShapeDtypeStruct((L * _D, Bt), jnp.float32),
        scratch_types=[
            pltpu.VMEM((_D * V,), jnp.float32),
            pltpu.VMEM((_LB, SB), jnp.int32),
            pltpu.VMEM((_NB, _D, SB), jnp.float32),
            pltpu.SemaphoreType.DMA,
            pltpu.SemaphoreType.DMA,
            pltpu.SemaphoreType.DMA,
            pltpu.SemaphoreType.DMA,
        ],
    )
    def gather_kernel(xt_hbm, tab_hbm, out_hbm, tab_v, idx_v, ob_v,
                      sem_o0, sem_o1, sem_o2, sem_o3):
        cid = lax.axis_index("c")
        sid = lax.axis_index("s")
        wid = sid * NC + cid
        b0 = pl.multiple_of(wid * SB, SB)
        out_sems = (sem_o0, sem_o1, sem_o2, sem_o3)

        pltpu.sync_copy(tab_hbm, tab_v)

        def stage_block(l0):
            pltpu.sync_copy(
                xt_hbm.at[pl.ds(l0, _LB), pl.ds(b0, SB)], idx_v)

        def compute_l(l, buf):
            ll = lax.rem(l, _LB)
            obuf = ob_v.at[buf]

            @plsc.parallel_loop(0, SB, step=16, unroll=4)
            def group(o):
                idx = idx_v[ll, pl.ds(o, 16)]
                for f in range(_D):
                    vals = plsc.load_gather(tab_v, [idx + (f * V)])
                    obuf[f, pl.ds(o, 16)] = vals
            pltpu.async_copy(
                ob_v.at[buf],
                out_hbm.at[pl.ds(pl.multiple_of(l * _D, _D), _D),
                           pl.ds(b0, SB)],
                out_sems[buf],
            )

        def drain_out(l, buf):
            pltpu.make_async_copy(
                ob_v.at[buf],
                out_hbm.at[pl.ds(pl.multiple_of(l * _D, _D), _D),
                           pl.ds(b0, SB)],
                out_sems[buf],
            ).wait()

        stage_block(0)
        for buf in range(_NB):
            compute_l(buf, buf)

        def body(i, carry):
            l0 = _NB * i

            @pl.when(lax.rem(l0, _LB) == 0)
            def _():
                stage_block(pl.multiple_of(l0, _LB))

            for buf in range(_NB):
                l = l0 + buf
                drain_out(l - _NB, buf)
                compute_l(l, buf)
            return carry

        lax.fori_loop(1, L // _NB, body, 0)
        for buf in range(_NB):
            drain_out(L - _NB + buf, buf)

    return gather_kernel


def kernel(x, wte, W, b):
    tableT = _fused_table_t(wte, W, b)
    Bt, L = x.shape
    info = plsc.get_sparse_core_info()
    xt = x.T.astype(jnp.int32)
    out = _make_gather(Bt, L, wte.shape[0], info.num_cores,
                       info.num_subcores)(xt, tableT.reshape(-1))
    return out.reshape(L, _D, Bt).transpose(2, 0, 1)


# speedup vs baseline: 1.0825x; 1.0012x over previous
import functools

import jax
import jax.numpy as jnp
from jax import lax
from jax.experimental import pallas as pl
from jax.experimental.pallas import tpu as pltpu
from jax.experimental.pallas import tpu_sc as plsc

_D = 16
_LB = 40
_NB = 4


def _table_body(wte_ref, w_ref, b_ref, out_ref):
    out_ref[...] = lax.dot_general(
        w_ref[...], wte_ref[...],
        (((1,), (1,)), ((), ())),
        preferred_element_type=jnp.float32,
    ) + b_ref[...]


def _fused_table_t(wte, W, b):
    v, d = wte.shape
    return pl.pallas_call(
        _table_body,
        out_shape=jax.ShapeDtypeStruct((d, v), jnp.float32),
    )(wte, W, b.reshape(d, 1))


@functools.cache
def _make_gather(Bt, L, V, NC, NS):
    NW = NC * NS
    SB = Bt // NW
    assert Bt % NW == 0 and L % _LB == 0 and SB % 16 == 0
    assert L % _NB == 0 and _LB % _NB == 0
    mesh = plsc.VectorSubcoreMesh(core_axis_name="c", subcore_axis_name="s")

    @functools.partial(
        pl.kernel,
        mesh=mesh,
        compiler_params=pltpu.CompilerParams(needs_layout_passes=False),
        out_type=jax.ShapeDtypeStruct((L * _D, Bt), jnp.float32),
        scratch_types=[
            pltpu.VMEM((_D * V,), jnp.float32),
            pltpu.VMEM((_LB, SB), jnp.int32),
            pltpu.VMEM((_NB, _D, SB), jnp.float32),
            pltpu.SemaphoreType.DMA,
            pltpu.SemaphoreType.DMA,
            pltpu.SemaphoreType.DMA,
            pltpu.SemaphoreType.DMA,
        ],
    )
    def gather_kernel(xt_hbm, tab_hbm, out_hbm, tab_v, idx_v, ob_v,
                      sem_o0, sem_o1, sem_o2, sem_o3):
        cid = lax.axis_index("c")
        sid = lax.axis_index("s")
        wid = sid * NC + cid
        b0 = pl.multiple_of(wid * SB, SB)
        out_sems = (sem_o0, sem_o1, sem_o2, sem_o3)

        pltpu.sync_copy(tab_hbm, tab_v)

        def stage_block(l0):
            pltpu.sync_copy(
                xt_hbm.at[pl.ds(l0, _LB), pl.ds(b0, SB)], idx_v)

        def compute_l(l, buf):
            ll = lax.rem(l, _LB)
            obuf = ob_v.at[buf]

            @plsc.parallel_loop(0, SB, step=16, unroll=4)
            def group(o):
                idx = idx_v[ll, pl.ds(o, 16)]
                for f in range(_D):
                    vals = plsc.load_gather(tab_v, [idx + (f * V)])
                    obuf[f, pl.ds(o, 16)] = vals
            pltpu.async_copy(
                ob_v.at[buf],
                out_hbm.at[pl.ds(pl.multiple_of(l * _D, _D), _D),
                           pl.ds(b0, SB)],
                out_sems[buf],
            )

        def drain_out(l, buf):
            pltpu.make_async_copy(
                ob_v.at[buf],
                out_hbm.at[pl.ds(pl.multiple_of(l * _D, _D), _D),
                           pl.ds(b0, SB)],
                out_sems[buf],
            ).wait()

        stage_block(0)
        for buf in range(_NB):
            compute_l(buf, buf)

        def body(i, carry):
            l0 = _NB * i

            @pl.when(lax.rem(l0, _LB) == 0)
            def _():
                stage_block(pl.multiple_of(l0, _LB))

            for buf in range(_NB):
                l = l0 + buf
                drain_out(l - _NB, buf)
                compute_l(l, buf)
            return carry

        lax.fori_loop(1, L // _NB, body, 0)
        for buf in range(_NB):
            drain_out(L - _NB + buf, buf)

    return gather_kernel


def kernel(x, wte, W, b):
    tableT = _fused_table_t(wte, W, b)
    Bt, L = x.shape
    info = plsc.get_sparse_core_info()
    xt = x.T.astype(jnp.int32)
    out = _make_gather(Bt, L, wte.shape[0], info.num_cores,
                       info.num_subcores)(xt, tableT.reshape(-1))
    return out.reshape(L, _D, Bt).transpose(2, 0, 1)
